# Initial kernel scaffold; baseline (speedup 1.0000x reference)
import jax
import jax.numpy as jnp
from jax.experimental import pallas as pl


def kernel(detection, cnn_feature, Wf1, bf1, Wf2, bf2, Wf3, bf3, Wf4, bf4, Wdh, bdh, Whm, bhm, Wdw, bdw, Wwh, bwh):
    raise NotImplementedError("write your pallas kernel here")


# speedup vs baseline: 20.8156x; 20.8156x over previous
import jax
import jax.numpy as jnp
from jax.experimental import pallas as pl
from jax.experimental.pallas import tpu as pltpu

ROI = 7
CLS_NUM = 2
NMS_T = 0.7
CT_T = 0.05
TOPK = 100
BLK = 256
BB = 25


def _nms_kernel(rows_ref, colsT_ref, keep_ref, supp_ref):
    npad = rows_ref.shape[1]
    nblk = npad // BLK
    iota_row = jax.lax.broadcasted_iota(jnp.int32, (1, BLK), 1)

    supp_ref[...] = jnp.zeros((1, npad), jnp.float32)

    def outer(b, _):
        base = b * BLK
        x1c = colsT_ref[pl.ds(base, BLK), 0:1]
        y1c = colsT_ref[pl.ds(base, BLK), 1:2]
        x2c = colsT_ref[pl.ds(base, BLK), 2:3]
        y2c = colsT_ref[pl.ds(base, BLK), 3:4]
        clsc = colsT_ref[pl.ds(base, BLK), 4:5]
        areac = (x2c - x1c) * (y2c - y1c)

        x1r = rows_ref[0:1, pl.ds(base, BLK)]
        y1r = rows_ref[1:2, pl.ds(base, BLK)]
        x2r = rows_ref[2:3, pl.ds(base, BLK)]
        y2r = rows_ref[3:4, pl.ds(base, BLK)]
        clsb = rows_ref[4:5, pl.ds(base, BLK)]
        arear = (x2r - x1r) * (y2r - y1r)

        xx1 = jnp.maximum(x1c, x1r)
        yy1 = jnp.maximum(y1c, y1r)
        xx2 = jnp.minimum(x2c, x2r)
        yy2 = jnp.minimum(y2c, y2r)
        inter = jnp.clip(xx2 - xx1, 0.0) * jnp.clip(yy2 - yy1, 0.0)
        iou = inter / (areac + arear - inter + 1e-9)

        def inner(i, _c):
            onei = iota_row == i
            oneif = jnp.where(onei, 1.0, 0.0)
            sb = supp_ref[0:1, pl.ds(base, BLK)]
            keep_i = jnp.sum(jnp.where(onei, sb, 0.0)) < 0.5
            cls_i = jnp.sum(jnp.where(onei, clsb, 0.0))
            rowf = jnp.dot(oneif, iou, preferred_element_type=jnp.float32)
            cond = (rowf > NMS_T) & (clsb == cls_i) & (iota_row > i) & keep_i
            supp_ref[0:1, pl.ds(base, BLK)] = jnp.where(cond, 1.0, sb)
            return _c

        jax.lax.fori_loop(0, BLK, inner, 0)

        keep_row = 1.0 - supp_ref[0:1, pl.ds(base, BLK)]

        def cross(cb, _c):
            cbase = cb * BLK
            X1 = rows_ref[0:1, pl.ds(cbase, BLK)]
            Y1 = rows_ref[1:2, pl.ds(cbase, BLK)]
            X2 = rows_ref[2:3, pl.ds(cbase, BLK)]
            Y2 = rows_ref[3:4, pl.ds(cbase, BLK)]
            CLS = rows_ref[4:5, pl.ds(cbase, BLK)]
            AREA = (X2 - X1) * (Y2 - Y1)
            cx1 = jnp.maximum(x1c, X1)
            cy1 = jnp.maximum(y1c, Y1)
            cx2 = jnp.minimum(x2c, X2)
            cy2 = jnp.minimum(y2c, Y2)
            cint = jnp.clip(cx2 - cx1, 0.0) * jnp.clip(cy2 - cy1, 0.0)
            ciou = cint / (areac + AREA - cint + 1e-9)
            ig = base + jax.lax.broadcasted_iota(jnp.int32, (BLK, BLK), 0)
            jg = cbase + jax.lax.broadcasted_iota(jnp.int32, (BLK, BLK), 1)
            condf = ((ciou > NMS_T) & (clsc == CLS) & (jg > ig)).astype(jnp.float32)
            add = jnp.dot(keep_row, condf, preferred_element_type=jnp.float32)
            old = supp_ref[0:1, pl.ds(cbase, BLK)]
            supp_ref[0:1, pl.ds(cbase, BLK)] = jnp.maximum(old, jnp.minimum(add, 1.0))
            return _c

        jax.lax.fori_loop(b, nblk, cross, 0)
        return _

    jax.lax.fori_loop(0, nblk, outer, 0)

    score = rows_ref[5:6, :]
    keep_ref[...] = ((supp_ref[...] < 0.5) & (score > CT_T)).astype(jnp.float32)


def _interp_mats(bx, n_lane):
    x1 = bx[:, 0:1]
    y1 = bx[:, 1:2]
    x2 = bx[:, 2:3]
    y2 = bx[:, 3:4]
    bw = (x2 - x1) / ROI
    bh = (y2 - y1) / ROI
    qi = jax.lax.broadcasted_iota(jnp.int32, (BB, ROI), 1).astype(jnp.float32)
    xs = x1 + (qi + 0.5) * bw
    ys = y1 + (qi + 0.5) * bh

    def mat(g):
        g0 = jnp.floor(g)
        d = g - g0
        i0 = jnp.clip(g0.astype(jnp.int32), 0, n_lane - 1)
        i1 = jnp.clip(i0 + 1, 0, n_lane - 1)
        lane = jax.lax.broadcasted_iota(jnp.int32, (BB, ROI, n_lane), 2)
        w = jnp.where(lane == i0[..., None], (1.0 - d)[..., None], 0.0)
        w = w + jnp.where(lane == i1[..., None], d[..., None], 0.0)
        return w

    return mat(ys), mat(xs)


def _conv3x3(x, w_ref, b_ref, cin, cout):
    zr = jnp.zeros((BB, 1, ROI, cin), jnp.float32)
    xp = jnp.concatenate([zr, x, zr], axis=1)
    zc = jnp.zeros((BB, ROI + 2, 1, cin), jnp.float32)
    xp = jnp.concatenate([zc, xp, zc], axis=2)
    acc = None
    for ki in range(3):
        for kj in range(3):
            patch = xp[:, ki:ki + ROI, kj:kj + ROI, :].reshape(BB * ROI * ROI, cin)
            term = jnp.dot(patch, w_ref[ki, kj], preferred_element_type=jnp.float32)
            acc = term if acc is None else acc + term
    y = jnp.maximum(acc + b_ref[...], 0.0)
    return y.reshape(BB, ROI, ROI, cout)


def _head_kernel(boxes_ref, featT_ref,
                 wf1_ref, bf1_ref, wf2_ref, bf2_ref, wf3_ref, bf3_ref,
                 wf4_ref, bf4_ref, wdh_ref, bdh_ref, whm_ref, bhm_ref,
                 wdw_ref, bdw_ref, wwh_ref, bwh_ref,
                 hm_ref, wh_ref):
    bx = boxes_ref[0]
    Wy, Wx = _interp_mats(bx, 128)
    tmp = jnp.dot(Wy.reshape(BB * ROI, 128), featT_ref[...],
                  preferred_element_type=jnp.float32)
    tmp = tmp.reshape(BB, ROI * 64, 128)
    out = jax.lax.dot_general(tmp, Wx, (((2,), (2,)), ((0,), (0,))),
                              preferred_element_type=jnp.float32)
    roi = out.reshape(BB, ROI, 64, ROI).transpose(0, 1, 3, 2)

    x = _conv3x3(roi, wf1_ref, bf1_ref, 64, 256)
    x = _conv3x3(x, wf2_ref, bf2_ref, 256, 256)
    x = _conv3x3(x, wf3_ref, bf3_ref, 256, 256)
    x = _conv3x3(x, wf4_ref, bf4_ref, 256, 256)
    xf = x.reshape(BB * ROI * ROI, 256)

    for di in range(2):
        for dj in range(2):
            t = di * 2 + dj
            h = jnp.dot(xf, wdh_ref[di, dj], preferred_element_type=jnp.float32)
            h = h + bdh_ref[...]
            hm = jnp.dot(h, whm_ref[...], preferred_element_type=jnp.float32)
            hm = hm + bhm_ref[0, 0]
            hm_ref[:, t] = hm.reshape(BB, ROI, ROI)
            w_ = jnp.dot(xf, wdw_ref[di, dj], preferred_element_type=jnp.float32)
            w_ = w_ + bdw_ref[...]
            wh = jnp.dot(w_, wwh_ref[...], preferred_element_type=jnp.float32)
            wh = wh + bwh_ref[...]
            wh_ref[:, t] = wh.reshape(BB, ROI, ROI, 2)


def kernel(detection, cnn_feature, Wf1, bf1, Wf2, bf2, Wf3, bf3, Wf4, bf4,
           Wdh, bdh, Whm, bhm, Wdw, bdw, Wwh, bwh):
    det = detection[0]
    n = det.shape[0]
    npad = ((n + BLK - 1) // BLK) * BLK
    scores = det[:, 4]

    order = jnp.argsort(-scores)
    dets_s = det[order]

    rows = jnp.zeros((8, npad), jnp.float32)
    rows = rows.at[0:4, :n].set(dets_s[:, 0:4].T)
    rows = rows.at[4, :n].set(dets_s[:, 5])
    rows = rows.at[4, n:].set(-1.0)
    rows = rows.at[5, :n].set(dets_s[:, 4])
    rows = rows.at[5, n:].set(-1e9)
    colsT = rows.T

    keep_s = pl.pallas_call(
        _nms_kernel,
        out_shape=jax.ShapeDtypeStruct((1, npad), jnp.float32),
        scratch_shapes=[pltpu.VMEM((1, npad), jnp.float32)],
    )(rows, colsT)

    keep_sorted = keep_s[0, :n] > 0.5
    keep = jnp.zeros((n,), bool).at[order].set(keep_sorted)
    sel = jnp.where(keep, scores, -1e9)
    _, top_idx = jax.lax.top_k(sel, TOPK)
    det_sel = det[top_idx]

    feat = cnn_feature[0]
    featT = feat.transpose(1, 0, 2).reshape(128, 64 * 128)
    boxes3 = det_sel[:, :4].reshape(TOPK // BB, BB, 4)

    wf1 = Wf1.transpose(2, 3, 1, 0)
    wf2 = Wf2.transpose(2, 3, 1, 0)
    wf3 = Wf3.transpose(2, 3, 1, 0)
    wf4 = Wf4.transpose(2, 3, 1, 0)
    wdh = jnp.flip(Wdh.transpose(2, 3, 1, 0), axis=(0, 1))
    wdw = jnp.flip(Wdw.transpose(2, 3, 1, 0), axis=(0, 1))
    whm = Whm[:, :, 0, 0].T
    wwh = Wwh[:, :, 0, 0].T

    full = lambda shp: pl.BlockSpec(shp, lambda i: (0,) * len(shp))
    nblk = TOPK // BB
    hm4, wh4 = pl.pallas_call(
        _head_kernel,
        grid=(nblk,),
        in_specs=[
            pl.BlockSpec((1, BB, 4), lambda i: (i, 0, 0)),
            full((128, 64 * 128)),
            full((3, 3, 64, 256)), full((1, 256)),
            full((3, 3, 256, 256)), full((1, 256)),
            full((3, 3, 256, 256)), full((1, 256)),
            full((3, 3, 256, 256)), full((1, 256)),
            full((2, 2, 256, 256)), full((1, 256)),
            full((256, 1)), full((1, 1)),
            full((2, 2, 256, 256)), full((1, 256)),
            full((256, 2)), full((1, 2)),
        ],
        out_specs=[
            pl.BlockSpec((BB, 4, ROI, ROI), lambda i: (i, 0, 0, 0)),
            pl.BlockSpec((BB, 4, ROI, ROI, 2), lambda i: (i, 0, 0, 0, 0)),
        ],
        out_shape=[
            jax.ShapeDtypeStruct((TOPK, 4, ROI, ROI), jnp.float32),
            jax.ShapeDtypeStruct((TOPK, 4, ROI, ROI, 2), jnp.float32),
        ],
    )(boxes3, featT,
      wf1, bf1.reshape(1, 256), wf2, bf2.reshape(1, 256),
      wf3, bf3.reshape(1, 256), wf4, bf4.reshape(1, 256),
      wdh, bdh.reshape(1, 256), whm, bhm.reshape(1, 1),
      wdw, bdw.reshape(1, 256), wwh, bwh.reshape(1, 2))

    hm = hm4.reshape(TOPK, 2, 2, ROI, ROI).transpose(0, 3, 1, 4, 2)
    cp_hm = hm.reshape(TOPK, 2 * ROI, 2 * ROI)[:, None, :, :]
    wh = wh4.reshape(TOPK, 2, 2, ROI, ROI, 2).transpose(0, 5, 3, 1, 4, 2)
    cp_wh = wh.reshape(TOPK, 2, 2 * ROI, 2 * ROI)
    return (det_sel, cp_hm, cp_wh)


# speedup vs baseline: 69.4608x; 3.3370x over previous
import jax
import jax.numpy as jnp
from jax.experimental import pallas as pl
from jax.experimental.pallas import tpu as pltpu

ROI = 7
CLS_NUM = 2
NMS_T = 0.7
CT_T = 0.05
TOPK = 100
BLK = 256
BB = 25


def _nms_kernel(rows_ref, colsT_ref, keep_ref, supp_ref):
    npad = rows_ref.shape[1]
    nblk = npad // BLK
    iota_row = jax.lax.broadcasted_iota(jnp.int32, (1, BLK), 1)

    supp_ref[...] = jnp.zeros((1, npad), jnp.float32)

    def outer(b, _):
        base = b * BLK
        x1c = colsT_ref[pl.ds(base, BLK), 0:1]
        y1c = colsT_ref[pl.ds(base, BLK), 1:2]
        x2c = colsT_ref[pl.ds(base, BLK), 2:3]
        y2c = colsT_ref[pl.ds(base, BLK), 3:4]
        clsc = colsT_ref[pl.ds(base, BLK), 4:5]
        areac = (x2c - x1c) * (y2c - y1c)

        x1r = rows_ref[0:1, pl.ds(base, BLK)]
        y1r = rows_ref[1:2, pl.ds(base, BLK)]
        x2r = rows_ref[2:3, pl.ds(base, BLK)]
        y2r = rows_ref[3:4, pl.ds(base, BLK)]
        clsb = rows_ref[4:5, pl.ds(base, BLK)]
        arear = (x2r - x1r) * (y2r - y1r)

        xx1 = jnp.maximum(x1c, x1r)
        yy1 = jnp.maximum(y1c, y1r)
        xx2 = jnp.minimum(x2c, x2r)
        yy2 = jnp.minimum(y2c, y2r)
        inter = jnp.clip(xx2 - xx1, 0.0) * jnp.clip(yy2 - yy1, 0.0)
        iou = inter / (areac + arear - inter + 1e-9)

        ii = jax.lax.broadcasted_iota(jnp.int32, (BLK, BLK), 0)
        jj = jax.lax.broadcasted_iota(jnp.int32, (BLK, BLK), 1)
        C = ((iou > NMS_T) & (clsc == clsb) & (jj > ii)).astype(jnp.float32)
        valid = 1.0 - supp_ref[0:1, pl.ds(base, BLK)]

        def fp_cond(c):
            K, Kprev, it = c
            return (it < BLK + 2) & jnp.any(K != Kprev)

        def fp_body(c):
            K, _, it = c
            hit = jnp.dot(K, C, preferred_element_type=jnp.float32)
            Knew = jnp.where(hit > 0.5, 0.0, valid)
            return (Knew, K, it + 1)

        keep_row, _, _ = jax.lax.while_loop(
            fp_cond, fp_body, (valid, valid - 1.0, 0))
        supp_ref[0:1, pl.ds(base, BLK)] = 1.0 - keep_row

        def cross(cb, _c):
            cbase = cb * BLK
            X1 = rows_ref[0:1, pl.ds(cbase, BLK)]
            Y1 = rows_ref[1:2, pl.ds(cbase, BLK)]
            X2 = rows_ref[2:3, pl.ds(cbase, BLK)]
            Y2 = rows_ref[3:4, pl.ds(cbase, BLK)]
            CLS = rows_ref[4:5, pl.ds(cbase, BLK)]
            AREA = (X2 - X1) * (Y2 - Y1)
            cx1 = jnp.maximum(x1c, X1)
            cy1 = jnp.maximum(y1c, Y1)
            cx2 = jnp.minimum(x2c, X2)
            cy2 = jnp.minimum(y2c, Y2)
            cint = jnp.clip(cx2 - cx1, 0.0) * jnp.clip(cy2 - cy1, 0.0)
            ciou = cint / (areac + AREA - cint + 1e-9)
            ig = base + jax.lax.broadcasted_iota(jnp.int32, (BLK, BLK), 0)
            jg = cbase + jax.lax.broadcasted_iota(jnp.int32, (BLK, BLK), 1)
            condf = ((ciou > NMS_T) & (clsc == CLS) & (jg > ig)).astype(jnp.float32)
            add = jnp.dot(keep_row, condf, preferred_element_type=jnp.float32)
            old = supp_ref[0:1, pl.ds(cbase, BLK)]
            supp_ref[0:1, pl.ds(cbase, BLK)] = jnp.maximum(old, jnp.minimum(add, 1.0))
            return _c

        jax.lax.fori_loop(b, nblk, cross, 0)
        return _

    jax.lax.fori_loop(0, nblk, outer, 0)

    score = rows_ref[5:6, :]
    keep_ref[...] = ((supp_ref[...] < 0.5) & (score > CT_T)).astype(jnp.float32)


def _interp_mats(bx, n_lane):
    x1 = bx[:, 0:1]
    y1 = bx[:, 1:2]
    x2 = bx[:, 2:3]
    y2 = bx[:, 3:4]
    bw = (x2 - x1) / ROI
    bh = (y2 - y1) / ROI
    qi = jax.lax.broadcasted_iota(jnp.int32, (BB, ROI), 1).astype(jnp.float32)
    xs = x1 + (qi + 0.5) * bw
    ys = y1 + (qi + 0.5) * bh

    def mat(g):
        g0 = jnp.floor(g)
        d = g - g0
        i0 = jnp.clip(g0.astype(jnp.int32), 0, n_lane - 1)
        i1 = jnp.clip(i0 + 1, 0, n_lane - 1)
        lane = jax.lax.broadcasted_iota(jnp.int32, (BB, ROI, n_lane), 2)
        w = jnp.where(lane == i0[..., None], (1.0 - d)[..., None], 0.0)
        w = w + jnp.where(lane == i1[..., None], d[..., None], 0.0)
        return w

    return mat(ys), mat(xs)


def _conv3x3(x, w_ref, b_ref, cin, cout):
    zr = jnp.zeros((BB, 1, ROI, cin), jnp.float32)
    xp = jnp.concatenate([zr, x, zr], axis=1)
    zc = jnp.zeros((BB, ROI + 2, 1, cin), jnp.float32)
    xp = jnp.concatenate([zc, xp, zc], axis=2)
    acc = None
    for ki in range(3):
        for kj in range(3):
            patch = xp[:, ki:ki + ROI, kj:kj + ROI, :].reshape(BB * ROI * ROI, cin)
            term = jnp.dot(patch, w_ref[ki, kj], preferred_element_type=jnp.float32)
            acc = term if acc is None else acc + term
    y = jnp.maximum(acc + b_ref[...], 0.0)
    return y.reshape(BB, ROI, ROI, cout)


def _head_kernel(boxes_ref, featT_ref,
                 wf1_ref, bf1_ref, wf2_ref, bf2_ref, wf3_ref, bf3_ref,
                 wf4_ref, bf4_ref, wdh_ref, bdh_ref, whm_ref, bhm_ref,
                 wdw_ref, bdw_ref, wwh_ref, bwh_ref,
                 hm_ref, wh_ref):
    bx = boxes_ref[0]
    Wy, Wx = _interp_mats(bx, 128)
    tmp = jnp.dot(Wy.reshape(BB * ROI, 128), featT_ref[...],
                  preferred_element_type=jnp.float32)
    tmp = tmp.reshape(BB, ROI * 64, 128)
    out = jax.lax.dot_general(tmp, Wx, (((2,), (2,)), ((0,), (0,))),
                              preferred_element_type=jnp.float32)
    roi = out.reshape(BB, ROI, 64, ROI).transpose(0, 1, 3, 2)

    x = _conv3x3(roi, wf1_ref, bf1_ref, 64, 256)
    x = _conv3x3(x, wf2_ref, bf2_ref, 256, 256)
    x = _conv3x3(x, wf3_ref, bf3_ref, 256, 256)
    x = _conv3x3(x, wf4_ref, bf4_ref, 256, 256)
    xf = x.reshape(BB * ROI * ROI, 256)

    for di in range(2):
        for dj in range(2):
            t = di * 2 + dj
            h = jnp.dot(xf, wdh_ref[di, dj], preferred_element_type=jnp.float32)
            h = h + bdh_ref[...]
            hm = jnp.dot(h, whm_ref[...], preferred_element_type=jnp.float32)
            hm = hm + bhm_ref[0, 0]
            hm_ref[:, t] = hm.reshape(BB, ROI, ROI)
            w_ = jnp.dot(xf, wdw_ref[di, dj], preferred_element_type=jnp.float32)
            w_ = w_ + bdw_ref[...]
            wh = jnp.dot(w_, wwh_ref[...], preferred_element_type=jnp.float32)
            wh = wh + bwh_ref[...]
            wh_ref[:, t] = wh.reshape(BB, ROI, ROI, 2)


def kernel(detection, cnn_feature, Wf1, bf1, Wf2, bf2, Wf3, bf3, Wf4, bf4,
           Wdh, bdh, Whm, bhm, Wdw, bdw, Wwh, bwh):
    det = detection[0]
    n = det.shape[0]
    npad = ((n + BLK - 1) // BLK) * BLK
    scores = det[:, 4]

    order = jnp.argsort(-scores)
    dets_s = det[order]

    rows = jnp.zeros((8, npad), jnp.float32)
    rows = rows.at[0:4, :n].set(dets_s[:, 0:4].T)
    rows = rows.at[4, :n].set(dets_s[:, 5])
    rows = rows.at[4, n:].set(-1.0)
    rows = rows.at[5, :n].set(dets_s[:, 4])
    rows = rows.at[5, n:].set(-1e9)
    colsT = rows.T

    keep_s = pl.pallas_call(
        _nms_kernel,
        out_shape=jax.ShapeDtypeStruct((1, npad), jnp.float32),
        scratch_shapes=[pltpu.VMEM((1, npad), jnp.float32)],
    )(rows, colsT)

    keep_sorted = keep_s[0, :n] > 0.5
    keep = jnp.zeros((n,), bool).at[order].set(keep_sorted)
    sel = jnp.where(keep, scores, -1e9)
    _, top_idx = jax.lax.top_k(sel, TOPK)
    det_sel = det[top_idx]

    feat = cnn_feature[0]
    featT = feat.transpose(1, 0, 2).reshape(128, 64 * 128)
    boxes3 = det_sel[:, :4].reshape(TOPK // BB, BB, 4)

    wf1 = Wf1.transpose(2, 3, 1, 0)
    wf2 = Wf2.transpose(2, 3, 1, 0)
    wf3 = Wf3.transpose(2, 3, 1, 0)
    wf4 = Wf4.transpose(2, 3, 1, 0)
    wdh = jnp.flip(Wdh.transpose(2, 3, 1, 0), axis=(0, 1))
    wdw = jnp.flip(Wdw.transpose(2, 3, 1, 0), axis=(0, 1))
    whm = Whm[:, :, 0, 0].T
    wwh = Wwh[:, :, 0, 0].T

    full = lambda shp: pl.BlockSpec(shp, lambda i: (0,) * len(shp))
    nblk = TOPK // BB
    hm4, wh4 = pl.pallas_call(
        _head_kernel,
        grid=(nblk,),
        in_specs=[
            pl.BlockSpec((1, BB, 4), lambda i: (i, 0, 0)),
            full((128, 64 * 128)),
            full((3, 3, 64, 256)), full((1, 256)),
            full((3, 3, 256, 256)), full((1, 256)),
            full((3, 3, 256, 256)), full((1, 256)),
            full((3, 3, 256, 256)), full((1, 256)),
            full((2, 2, 256, 256)), full((1, 256)),
            full((256, 1)), full((1, 1)),
            full((2, 2, 256, 256)), full((1, 256)),
            full((256, 2)), full((1, 2)),
        ],
        out_specs=[
            pl.BlockSpec((BB, 4, ROI, ROI), lambda i: (i, 0, 0, 0)),
            pl.BlockSpec((BB, 4, ROI, ROI, 2), lambda i: (i, 0, 0, 0, 0)),
        ],
        out_shape=[
            jax.ShapeDtypeStruct((TOPK, 4, ROI, ROI), jnp.float32),
            jax.ShapeDtypeStruct((TOPK, 4, ROI, ROI, 2), jnp.float32),
        ],
    )(boxes3, featT,
      wf1, bf1.reshape(1, 256), wf2, bf2.reshape(1, 256),
      wf3, bf3.reshape(1, 256), wf4, bf4.reshape(1, 256),
      wdh, bdh.reshape(1, 256), whm, bhm.reshape(1, 1),
      wdw, bdw.reshape(1, 256), wwh, bwh.reshape(1, 2))

    hm = hm4.reshape(TOPK, 2, 2, ROI, ROI).transpose(0, 3, 1, 4, 2)
    cp_hm = hm.reshape(TOPK, 2 * ROI, 2 * ROI)[:, None, :, :]
    wh = wh4.reshape(TOPK, 2, 2, ROI, ROI, 2).transpose(0, 5, 3, 1, 4, 2)
    cp_wh = wh.reshape(TOPK, 2, 2 * ROI, 2 * ROI)
    return (det_sel, cp_hm, cp_wh)
